# baseline (speedup 1.0000x reference)
import functools

import jax
import jax.numpy as jnp
from jax import lax
from jax.experimental import pallas as pl
from jax.experimental.pallas import tpu as pltpu
from jax.experimental.pallas import tpu_sc as plsc

_NUM_CORES = 2
_NUM_SUBCORES = 16
_CHUNK = 400
_LANES = 16
_CNTW = 8


@functools.lru_cache(maxsize=None)
def _sc_aggregate(n, d, e):
    dh = d // 2
    per_tile = e // _NUM_SUBCORES
    chunks = per_tile // _CHUNK
    half_chunks = chunks // 2
    groups = _CHUNK // _LANES
    rows_per_sub = n // _NUM_SUBCORES

    mesh = plsc.VectorSubcoreMesh(core_axis_name="c", subcore_axis_name="s",
                                  num_cores=_NUM_CORES,
                                  num_subcores=_NUM_SUBCORES)

    @functools.partial(
        pl.kernel,
        compiler_params=pltpu.CompilerParams(use_tc_tiling_on_sc=False,
                                             needs_layout_passes=False),
        out_type=[
            jax.ShapeDtypeStruct((n, d), jnp.float32),
            jax.ShapeDtypeStruct((_NUM_CORES, n, _CNTW), jnp.float32),
        ],
        mesh=mesh,
        scratch_types=[
            pltpu.VMEM((2, 2 * _CHUNK), jnp.int32),
            pltpu.VMEM((2, 2 * _CHUNK), jnp.int32),
            pltpu.VMEM((2, _CHUNK), jnp.int32),
            pltpu.VMEM((2, _CHUNK), jnp.int32),
            pltpu.VMEM((2, _CHUNK, dh), jnp.float32),
            pltpu.VMEM((_CHUNK, _CNTW), jnp.float32),
            pltpu.VMEM_SHARED((n, dh), jnp.float32),
            pltpu.VMEM_SHARED((n, _CNTW), jnp.float32),
            pltpu.SemaphoreType.DMA,
            pltpu.SemaphoreType.DMA,
        ],
    )
    def agg(xr_hbm, sp_hbm, dp_hbm, zrow_hbm, zcnt_hbm, ones_hbm,
            part_hbm, cnt_hbm, sp_v, dp_v, src_v, dst_v, rows_v, ones_v,
            acc_sh, cnt_sh, gsem0, gsem1):
        i32 = lambda v: jnp.int32(v)
        c = lax.convert_element_type(lax.axis_index("c"), jnp.int32)
        s = lax.convert_element_type(lax.axis_index("s"), jnp.int32)
        row0 = s * i32(rows_per_sub)

        pltpu.sync_copy(zrow_hbm, acc_sh.at[pl.ds(row0, rows_per_sub)])
        pltpu.sync_copy(zcnt_hbm, cnt_sh.at[pl.ds(row0, rows_per_sub)])
        pltpu.sync_copy(ones_hbm, ones_v)
        plsc.subcore_barrier()

        ebase = s * i32(per_tile)
        even = lax.iota(jnp.int32, _LANES) * 2
        bufs = (
            (sp_v.at[0], dp_v.at[0], src_v.at[0], dst_v.at[0], rows_v.at[0],
             gsem0),
            (sp_v.at[1], dp_v.at[1], src_v.at[1], dst_v.at[1], rows_v.at[1],
             gsem1),
        )

        def load_and_gather(chunk_i, k):
            spv, dpv, sv, dv, rv, sem = bufs[k]
            poff = (ebase + chunk_i * i32(_CHUNK)) * 2
            pltpu.sync_copy(sp_hbm.at[pl.ds(poff, 2 * _CHUNK)], spv)
            pltpu.sync_copy(dp_hbm.at[pl.ds(poff, 2 * _CHUNK)], dpv)
            for g in range(groups):
                pos = even + i32(2 * _LANES * g)
                sl = pl.ds(g * _LANES, _LANES)
                svals = plsc.load_gather(spv, [pos])
                sv[sl] = svals + svals + c
                dv[sl] = plsc.load_gather(dpv, [pos])
            pltpu.async_copy(xr_hbm.at[sv], rv, sem)

        def drain_and_scatter(chunk_i, k):
            spv, dpv, sv, dv, rv, sem = bufs[k]
            pltpu.make_async_copy(xr_hbm.at[sv], rv, sem).wait()
            pltpu.sync_copy(rv, acc_sh.at[dv], add=True)
            count_here = (chunk_i < i32(half_chunks)) == (c == 0)

            @pl.when(count_here)
            def _():
                pltpu.sync_copy(ones_v, cnt_sh.at[dv], add=True)

        load_and_gather(i32(0), 0)
        load_and_gather(i32(1), 1)

        @pl.loop(0, half_chunks)
        def _(t):
            t = lax.convert_element_type(t, jnp.int32)
            a = t * i32(2)
            drain_and_scatter(a, 0)

            @pl.when(a + i32(2) < i32(chunks))
            def _():
                load_and_gather(a + i32(2), 0)

            drain_and_scatter(a + i32(1), 1)

            @pl.when(a + i32(3) < i32(chunks))
            def _():
                load_and_gather(a + i32(3), 1)

        plsc.subcore_barrier()
        pltpu.sync_copy(acc_sh.at[pl.ds(row0, rows_per_sub)],
                        part_hbm.at[pl.ds(row0, rows_per_sub),
                                    pl.ds(c * i32(dh), dh)])
        pltpu.sync_copy(cnt_sh.at[pl.ds(row0, rows_per_sub)],
                        cnt_hbm.at[c, pl.ds(row0, rows_per_sub)])

    return agg


def _tc_self_body(x_ref, ws_ref, b_ref, o_ref):
    dn = (((1,), (0,)), ((), ()))
    o_ref[...] = lax.dot_general(x_ref[...], ws_ref[...], dn,
                                 precision=lax.Precision.HIGHEST,
                                 preferred_element_type=jnp.float32) + b_ref[...]


def _tc_finish_body(xw_ref, p_ref, c_ref, wn_ref, o_ref):
    deg = jnp.maximum(c_ref[0, :, 0:1] + c_ref[1, :, 0:1], 1.0)
    h = p_ref[...] / deg
    dn = (((1,), (0,)), ((), ()))
    acc = lax.dot_general(h, wn_ref[...], dn, precision=lax.Precision.HIGHEST,
                          preferred_element_type=jnp.float32)
    o_ref[...] = jnp.maximum(acc + xw_ref[...], 0.0)


def kernel(x, edge_index, W_self, W_neigh, b):
    with jax.enable_x64(False):
        return _kernel_32(x, edge_index, W_self, W_neigh, b)


def _kernel_32(x, edge_index, W_self, W_neigh, b):
    n, d = x.shape
    e = edge_index.shape[1]
    dh = d // 2

    ei32 = lax.bitcast_convert_type(edge_index, jnp.int32)
    sp = ei32[0].reshape(2 * e)
    dp = ei32[1].reshape(2 * e)

    xr = jnp.reshape(x, (2 * n, dh))
    rows_per_sub = n // _NUM_SUBCORES
    zrow = jnp.zeros((rows_per_sub, dh), jnp.float32)
    zcnt = jnp.zeros((rows_per_sub, _CNTW), jnp.float32)
    ones = jnp.ones((_CHUNK, _CNTW), jnp.float32)

    part, cnt = _sc_aggregate(n, d, e)(xr, sp, dp, zrow, zcnt, ones)

    bl = 1000
    grid = (n // bl,)
    xw = pl.pallas_call(
        _tc_self_body,
        grid=grid,
        in_specs=[
            pl.BlockSpec((bl, d), lambda i: (i, 0)),
            pl.BlockSpec((d, d), lambda i: (0, 0)),
            pl.BlockSpec((1, d), lambda i: (0, 0)),
        ],
        out_specs=pl.BlockSpec((bl, d), lambda i: (i, 0)),
        out_shape=jax.ShapeDtypeStruct((n, d), jnp.float32),
    )(x, W_self, b.reshape(1, d).astype(jnp.float32))

    out = pl.pallas_call(
        _tc_finish_body,
        grid=grid,
        in_specs=[
            pl.BlockSpec((bl, d), lambda i: (i, 0)),
            pl.BlockSpec((bl, d), lambda i: (i, 0)),
            pl.BlockSpec((_NUM_CORES, bl, _CNTW), lambda i: (0, i, 0)),
            pl.BlockSpec((d, d), lambda i: (0, 0)),
        ],
        out_specs=pl.BlockSpec((bl, d), lambda i: (i, 0)),
        out_shape=jax.ShapeDtypeStruct((n, d), jnp.float32),
    )(xw, part, cnt, W_neigh)
    return out


# speedup vs baseline: 2.7341x; 2.7341x over previous
import functools

import jax
import jax.numpy as jnp
from jax import lax
from jax.experimental import pallas as pl
from jax.experimental.pallas import tpu as pltpu
from jax.experimental.pallas import tpu_sc as plsc

_NUM_CORES = 2
_NUM_SUBCORES = 16
_CHUNK = 400


@functools.lru_cache(maxsize=None)
def _sc_aggregate(n, d, e):
    dh = d // 2
    per_tile = e // _NUM_SUBCORES
    chunks = per_tile // _CHUNK
    half_chunks = chunks // 2
    rows_per_sub = n // _NUM_SUBCORES

    mesh = plsc.VectorSubcoreMesh(core_axis_name="c", subcore_axis_name="s",
                                  num_cores=_NUM_CORES,
                                  num_subcores=_NUM_SUBCORES)

    @functools.partial(
        pl.kernel,
        compiler_params=pltpu.CompilerParams(use_tc_tiling_on_sc=False),
        out_type=[
            jax.ShapeDtypeStruct((n, d), jnp.float32),
            jax.ShapeDtypeStruct((_NUM_CORES, n, 8), jnp.float32),
        ],
        mesh=mesh,
        scratch_types=[
            pltpu.VMEM((2, _CHUNK), jnp.int32),
            pltpu.VMEM((2, _CHUNK), jnp.int32),
            pltpu.VMEM((2, _CHUNK, dh), jnp.float32),
            pltpu.VMEM((_CHUNK, 8), jnp.float32),
            pltpu.VMEM_SHARED((n, dh), jnp.float32),
            pltpu.VMEM_SHARED((n, 8), jnp.float32),
            pltpu.SemaphoreType.DMA,
            pltpu.SemaphoreType.DMA,
        ],
    )
    def agg(xr_hbm, src2_hbm, dst_hbm, zrow_hbm, zcnt_hbm, ones_hbm,
            part_hbm, cnt_hbm, src_v, dst_v, rows_v, ones_v, acc_sh, cnt_sh,
            gsem0, gsem1):
        i32 = lambda v: jnp.int32(v)
        c = lax.convert_element_type(lax.axis_index("c"), jnp.int32)
        s = lax.convert_element_type(lax.axis_index("s"), jnp.int32)
        row0 = s * i32(rows_per_sub)

        pltpu.sync_copy(zrow_hbm, acc_sh.at[pl.ds(row0, rows_per_sub)])
        pltpu.sync_copy(zcnt_hbm, cnt_sh.at[pl.ds(row0, rows_per_sub)])
        pltpu.sync_copy(ones_hbm, ones_v)
        plsc.subcore_barrier()

        ebase = s * i32(per_tile)
        bufs = (
            (src_v.at[0], dst_v.at[0], rows_v.at[0], gsem0),
            (src_v.at[1], dst_v.at[1], rows_v.at[1], gsem1),
        )

        def load_and_gather(chunk_i, k):
            sv, dv, rv, sem = bufs[k]
            off = ebase + chunk_i * i32(_CHUNK)
            pltpu.sync_copy(src2_hbm.at[c, pl.ds(off, _CHUNK)], sv)
            pltpu.sync_copy(dst_hbm.at[pl.ds(off, _CHUNK)], dv)
            pltpu.async_copy(xr_hbm.at[sv], rv, sem)

        def drain_and_scatter(chunk_i, k):
            sv, dv, rv, sem = bufs[k]
            pltpu.make_async_copy(xr_hbm.at[sv], rv, sem).wait()
            pltpu.sync_copy(rv, acc_sh.at[dv], add=True)
            count_here = (chunk_i < i32(half_chunks)) == (c == 0)

            @pl.when(count_here)
            def _():
                pltpu.sync_copy(ones_v, cnt_sh.at[dv], add=True)

        load_and_gather(i32(0), 0)
        load_and_gather(i32(1), 1)

        @pl.loop(0, half_chunks)
        def _(t):
            t = lax.convert_element_type(t, jnp.int32)
            a = t * i32(2)
            drain_and_scatter(a, 0)

            @pl.when(a + i32(2) < i32(chunks))
            def _():
                load_and_gather(a + i32(2), 0)

            drain_and_scatter(a + i32(1), 1)

            @pl.when(a + i32(3) < i32(chunks))
            def _():
                load_and_gather(a + i32(3), 1)

        plsc.subcore_barrier()
        pltpu.sync_copy(acc_sh.at[pl.ds(row0, rows_per_sub)],
                        part_hbm.at[pl.ds(row0, rows_per_sub),
                                    pl.ds(c * i32(dh), dh)])
        pltpu.sync_copy(cnt_sh.at[pl.ds(row0, rows_per_sub)],
                        cnt_hbm.at[c, pl.ds(row0, rows_per_sub)])

    return agg


def _tc_self_body(x_ref, ws_ref, b_ref, o_ref):
    hp = lax.Precision.HIGHEST
    dn = (((1,), (0,)), ((), ()))
    o_ref[...] = lax.dot_general(x_ref[...], ws_ref[...], dn, precision=hp,
                                 preferred_element_type=jnp.float32) + b_ref[...]


def _tc_finish_body(xw_ref, p_ref, c_ref, wn_ref, o_ref):
    deg = jnp.maximum(c_ref[0, :, 0:1] + c_ref[1, :, 0:1], 1.0)
    h = p_ref[...] / deg
    dn = (((1,), (0,)), ((), ()))
    acc = lax.dot_general(h, wn_ref[...], dn, precision=lax.Precision.HIGHEST,
                          preferred_element_type=jnp.float32)
    o_ref[...] = jnp.maximum(acc + xw_ref[...], 0.0)


def kernel(x, edge_index, W_self, W_neigh, b):
    with jax.enable_x64(False):
        return _kernel_32(x, edge_index, W_self, W_neigh, b)


def _kernel_32(x, edge_index, W_self, W_neigh, b):
    n, d = x.shape
    e = edge_index.shape[1]
    dh = d // 2

    ei32 = lax.bitcast_convert_type(edge_index, jnp.int32)
    src = ei32[0, :, 0]
    dst = ei32[1, :, 0]
    src2 = jnp.stack([2 * src, 2 * src + 1])

    xr = jnp.reshape(x, (2 * n, dh))
    rows_per_sub = n // _NUM_SUBCORES
    zrow = jnp.zeros((rows_per_sub, dh), jnp.float32)
    zcnt = jnp.zeros((rows_per_sub, 8), jnp.float32)
    ones = jnp.ones((_CHUNK, 8), jnp.float32)

    part, cnt = _sc_aggregate(n, d, e)(xr, src2, dst, zrow, zcnt, ones)

    bl = 1000
    grid = (n // bl,)
    xw = pl.pallas_call(
        _tc_self_body,
        grid=grid,
        in_specs=[
            pl.BlockSpec((bl, d), lambda i: (i, 0)),
            pl.BlockSpec((d, d), lambda i: (0, 0)),
            pl.BlockSpec((1, d), lambda i: (0, 0)),
        ],
        out_specs=pl.BlockSpec((bl, d), lambda i: (i, 0)),
        out_shape=jax.ShapeDtypeStruct((n, d), jnp.float32),
    )(x, W_self, b.reshape(1, d).astype(jnp.float32))

    out = pl.pallas_call(
        _tc_finish_body,
        grid=grid,
        in_specs=[
            pl.BlockSpec((bl, d), lambda i: (i, 0)),
            pl.BlockSpec((bl, d), lambda i: (i, 0)),
            pl.BlockSpec((_NUM_CORES, bl, 8), lambda i: (0, i, 0)),
            pl.BlockSpec((d, d), lambda i: (0, 0)),
        ],
        out_specs=pl.BlockSpec((bl, d), lambda i: (i, 0)),
        out_shape=jax.ShapeDtypeStruct((n, d), jnp.float32),
    )(xw, part, cnt, W_neigh)
    return out


# speedup vs baseline: 2.7725x; 1.0141x over previous
import functools

import jax
import jax.numpy as jnp
from jax import lax
from jax.experimental import pallas as pl
from jax.experimental.pallas import tpu as pltpu
from jax.experimental.pallas import tpu_sc as plsc

_NUM_CORES = 2
_NUM_SUBCORES = 16
_CHUNK = 400


@functools.lru_cache(maxsize=None)
def _sc_aggregate(n, d, e):
    dh = d // 2
    per_tile = e // _NUM_SUBCORES
    chunks = per_tile // _CHUNK
    half_chunks = chunks // 2
    rows_per_sub = n // _NUM_SUBCORES

    mesh = plsc.VectorSubcoreMesh(core_axis_name="c", subcore_axis_name="s",
                                  num_cores=_NUM_CORES,
                                  num_subcores=_NUM_SUBCORES)

    @functools.partial(
        pl.kernel,
        compiler_params=pltpu.CompilerParams(use_tc_tiling_on_sc=False),
        out_type=[
            jax.ShapeDtypeStruct((n, d), jnp.float32),
            jax.ShapeDtypeStruct((_NUM_CORES, n, 8), jnp.float32),
        ],
        mesh=mesh,
        scratch_types=[
            pltpu.VMEM((2, _CHUNK), jnp.int32),
            pltpu.VMEM((2, _CHUNK), jnp.int32),
            pltpu.VMEM((2, _CHUNK, dh), jnp.float32),
            pltpu.VMEM((_CHUNK, 8), jnp.float32),
            pltpu.VMEM_SHARED((n, dh), jnp.float32),
            pltpu.VMEM_SHARED((n, 8), jnp.float32),
            pltpu.SemaphoreType.DMA,
            pltpu.SemaphoreType.DMA,
        ],
    )
    def agg(xr_hbm, src2_hbm, dst_hbm, zrow_hbm, zcnt_hbm, ones_hbm,
            part_hbm, cnt_hbm, src_v, dst_v, rows_v, ones_v, acc_sh, cnt_sh,
            gsem0, gsem1):
        i32 = lambda v: jnp.int32(v)
        c = lax.convert_element_type(lax.axis_index("c"), jnp.int32)
        s = lax.convert_element_type(lax.axis_index("s"), jnp.int32)
        row0 = s * i32(rows_per_sub)

        pltpu.sync_copy(zrow_hbm, acc_sh.at[pl.ds(row0, rows_per_sub)])
        pltpu.sync_copy(zcnt_hbm, cnt_sh.at[pl.ds(row0, rows_per_sub)])
        pltpu.sync_copy(ones_hbm, ones_v)
        plsc.subcore_barrier()

        ebase = s * i32(per_tile)
        bufs = (
            (src_v.at[0], dst_v.at[0], rows_v.at[0], gsem0),
            (src_v.at[1], dst_v.at[1], rows_v.at[1], gsem1),
        )

        def load_and_gather(chunk_i, k):
            sv, dv, rv, sem = bufs[k]
            off = ebase + chunk_i * i32(_CHUNK)
            pltpu.sync_copy(src2_hbm.at[c, pl.ds(off, _CHUNK)], sv)
            pltpu.sync_copy(dst_hbm.at[pl.ds(off, _CHUNK)], dv)
            pltpu.async_copy(xr_hbm.at[sv], rv, sem)

        def drain_and_scatter(chunk_i, k):
            sv, dv, rv, sem = bufs[k]
            pltpu.make_async_copy(xr_hbm.at[sv], rv, sem).wait()
            pltpu.sync_copy(rv, acc_sh.at[dv], add=True)
            count_here = (chunk_i < i32(half_chunks)) == (c == 0)

            @pl.when(count_here)
            def _():
                pltpu.sync_copy(ones_v, cnt_sh.at[dv], add=True)

        load_and_gather(i32(0), 0)
        load_and_gather(i32(1), 1)

        @pl.loop(0, half_chunks)
        def _(t):
            t = lax.convert_element_type(t, jnp.int32)
            a = t * i32(2)
            drain_and_scatter(a, 0)

            @pl.when(a + i32(2) < i32(chunks))
            def _():
                load_and_gather(a + i32(2), 0)

            drain_and_scatter(a + i32(1), 1)

            @pl.when(a + i32(3) < i32(chunks))
            def _():
                load_and_gather(a + i32(3), 1)

        plsc.subcore_barrier()
        pltpu.sync_copy(acc_sh.at[pl.ds(row0, rows_per_sub)],
                        part_hbm.at[pl.ds(row0, rows_per_sub),
                                    pl.ds(c * i32(dh), dh)])
        pltpu.sync_copy(cnt_sh.at[pl.ds(row0, rows_per_sub)],
                        cnt_hbm.at[c, pl.ds(row0, rows_per_sub)])

    return agg


def _tc_self_body(x_ref, ws_ref, b_ref, o_ref):
    hp = lax.Precision.DEFAULT
    dn = (((1,), (0,)), ((), ()))
    o_ref[...] = lax.dot_general(x_ref[...], ws_ref[...], dn, precision=hp,
                                 preferred_element_type=jnp.float32) + b_ref[...]


def _tc_finish_body(xw_ref, p_ref, c_ref, wn_ref, o_ref):
    deg = jnp.maximum(c_ref[0, :, 0:1] + c_ref[1, :, 0:1], 1.0)
    h = p_ref[...] / deg
    dn = (((1,), (0,)), ((), ()))
    acc = lax.dot_general(h, wn_ref[...], dn, precision=lax.Precision.DEFAULT,
                          preferred_element_type=jnp.float32)
    o_ref[...] = jnp.maximum(acc + xw_ref[...], 0.0)


def kernel(x, edge_index, W_self, W_neigh, b):
    with jax.enable_x64(False):
        return _kernel_32(x, edge_index, W_self, W_neigh, b)


def _kernel_32(x, edge_index, W_self, W_neigh, b):
    n, d = x.shape
    e = edge_index.shape[1]
    dh = d // 2

    ei32 = lax.bitcast_convert_type(edge_index, jnp.int32)
    src = ei32[0, :, 0]
    dst = ei32[1, :, 0]
    src2 = jnp.stack([2 * src, 2 * src + 1])

    xr = jnp.reshape(x, (2 * n, dh))
    rows_per_sub = n // _NUM_SUBCORES
    zrow = jnp.zeros((rows_per_sub, dh), jnp.float32)
    zcnt = jnp.zeros((rows_per_sub, 8), jnp.float32)
    ones = jnp.ones((_CHUNK, 8), jnp.float32)

    part, cnt = _sc_aggregate(n, d, e)(xr, src2, dst, zrow, zcnt, ones)

    bl = 1000
    grid = (n // bl,)
    xw = pl.pallas_call(
        _tc_self_body,
        grid=grid,
        in_specs=[
            pl.BlockSpec((bl, d), lambda i: (i, 0)),
            pl.BlockSpec((d, d), lambda i: (0, 0)),
            pl.BlockSpec((1, d), lambda i: (0, 0)),
        ],
        out_specs=pl.BlockSpec((bl, d), lambda i: (i, 0)),
        out_shape=jax.ShapeDtypeStruct((n, d), jnp.float32),
    )(x, W_self, b.reshape(1, d).astype(jnp.float32))

    out = pl.pallas_call(
        _tc_finish_body,
        grid=grid,
        in_specs=[
            pl.BlockSpec((bl, d), lambda i: (i, 0)),
            pl.BlockSpec((bl, d), lambda i: (i, 0)),
            pl.BlockSpec((_NUM_CORES, bl, 8), lambda i: (0, i, 0)),
            pl.BlockSpec((d, d), lambda i: (0, 0)),
        ],
        out_specs=pl.BlockSpec((bl, d), lambda i: (i, 0)),
        out_shape=jax.ShapeDtypeStruct((n, d), jnp.float32),
    )(xw, part, cnt, W_neigh)
    return out


# speedup vs baseline: 2.7771x; 1.0016x over previous
import functools

import jax
import jax.numpy as jnp
from jax import lax
from jax.experimental import pallas as pl
from jax.experimental.pallas import tpu as pltpu
from jax.experimental.pallas import tpu_sc as plsc

_NUM_CORES = 2
_NUM_SUBCORES = 16
_CHUNK = 400


@functools.lru_cache(maxsize=None)
def _sc_aggregate(n, d, e):
    dh = d // 2
    per_tile = e // _NUM_SUBCORES
    chunks = per_tile // _CHUNK
    half_chunks = chunks // 2
    rows_per_sub = n // _NUM_SUBCORES

    mesh = plsc.VectorSubcoreMesh(core_axis_name="c", subcore_axis_name="s",
                                  num_cores=_NUM_CORES,
                                  num_subcores=_NUM_SUBCORES)

    @functools.partial(
        pl.kernel,
        compiler_params=pltpu.CompilerParams(use_tc_tiling_on_sc=False),
        out_type=[
            jax.ShapeDtypeStruct((n, d), jnp.float32),
            jax.ShapeDtypeStruct((_NUM_CORES, n, 8), jnp.float32),
        ],
        mesh=mesh,
        scratch_types=[
            pltpu.VMEM((2, _CHUNK), jnp.int32),
            pltpu.VMEM((2, _CHUNK), jnp.int32),
            pltpu.VMEM((2, _CHUNK, dh), jnp.float32),
            pltpu.VMEM((_CHUNK, 8), jnp.float32),
            pltpu.VMEM_SHARED((n, dh), jnp.float32),
            pltpu.VMEM_SHARED((n, 8), jnp.float32),
            pltpu.SemaphoreType.DMA,
            pltpu.SemaphoreType.DMA,
        ],
    )
    def agg(xr_hbm, src2_hbm, dst_hbm, zrow_hbm, zcnt_hbm, ones_hbm,
            part_hbm, cnt_hbm, src_v, dst_v, rows_v, ones_v, acc_sh, cnt_sh,
            gsem0, gsem1):
        i32 = lambda v: jnp.int32(v)
        c = lax.convert_element_type(lax.axis_index("c"), jnp.int32)
        s = lax.convert_element_type(lax.axis_index("s"), jnp.int32)
        row0 = s * i32(rows_per_sub)

        pltpu.sync_copy(zrow_hbm, acc_sh.at[pl.ds(row0, rows_per_sub)])
        pltpu.sync_copy(zcnt_hbm, cnt_sh.at[pl.ds(row0, rows_per_sub)])
        pltpu.sync_copy(ones_hbm, ones_v)
        plsc.subcore_barrier()

        ebase = s * i32(per_tile)
        bufs = (
            (src_v.at[0], dst_v.at[0], rows_v.at[0], gsem0),
            (src_v.at[1], dst_v.at[1], rows_v.at[1], gsem1),
        )

        def load_and_gather(chunk_i, k):
            sv, dv, rv, sem = bufs[k]
            off = ebase + chunk_i * i32(_CHUNK)
            pltpu.sync_copy(src2_hbm.at[c, pl.ds(off, _CHUNK)], sv)
            pltpu.sync_copy(dst_hbm.at[pl.ds(off, _CHUNK)], dv)
            pltpu.async_copy(xr_hbm.at[sv], rv, sem)

        def drain_and_scatter(chunk_i, k):
            sv, dv, rv, sem = bufs[k]
            pltpu.make_async_copy(xr_hbm.at[sv], rv, sem).wait()
            pltpu.sync_copy(rv, acc_sh.at[dv], add=True)
            count_here = (chunk_i < i32(half_chunks)) == (c == 0)

            @pl.when(count_here)
            def _():
                pltpu.sync_copy(ones_v, cnt_sh.at[dv], add=True)

        load_and_gather(i32(0), 0)
        load_and_gather(i32(1), 1)

        @pl.loop(0, half_chunks)
        def _(t):
            t = lax.convert_element_type(t, jnp.int32)
            a = t * i32(2)
            drain_and_scatter(a, 0)

            @pl.when(a + i32(2) < i32(chunks))
            def _():
                load_and_gather(a + i32(2), 0)

            drain_and_scatter(a + i32(1), 1)

            @pl.when(a + i32(3) < i32(chunks))
            def _():
                load_and_gather(a + i32(3), 1)

        plsc.subcore_barrier()
        pltpu.sync_copy(acc_sh.at[pl.ds(row0, rows_per_sub)],
                        part_hbm.at[pl.ds(row0, rows_per_sub),
                                    pl.ds(c * i32(dh), dh)])
        pltpu.sync_copy(cnt_sh.at[pl.ds(row0, rows_per_sub)],
                        cnt_hbm.at[c, pl.ds(row0, rows_per_sub)])

    return agg


def _tc_self_body(x_ref, ws_ref, b_ref, o_ref):
    hp = lax.Precision.DEFAULT
    dn = (((1,), (0,)), ((), ()))
    o_ref[...] = lax.dot_general(x_ref[...], ws_ref[...], dn, precision=hp,
                                 preferred_element_type=jnp.float32) + b_ref[...]


def _tc_finish_body(xw_ref, p_ref, c_ref, wn_ref, o_ref):
    deg = jnp.maximum(c_ref[0, :, 0:1] + c_ref[1, :, 0:1], 1.0)
    h = p_ref[...] / deg
    dn = (((1,), (0,)), ((), ()))
    acc = lax.dot_general(h, wn_ref[...], dn, precision=lax.Precision.DEFAULT,
                          preferred_element_type=jnp.float32)
    o_ref[...] = jnp.maximum(acc + xw_ref[...], 0.0)


def kernel(x, edge_index, W_self, W_neigh, b):
    with jax.enable_x64(False):
        return _kernel_32(x, edge_index, W_self, W_neigh, b)


def _kernel_32(x, edge_index, W_self, W_neigh, b):
    n, d = x.shape
    e = edge_index.shape[1]
    dh = d // 2

    ei32 = lax.bitcast_convert_type(edge_index, jnp.int32)
    eit = jnp.transpose(ei32, (0, 2, 1))
    src = eit[0, 0]
    dst = eit[1, 0]
    src2 = jnp.stack([2 * src, 2 * src + 1])

    xr = jnp.reshape(x, (2 * n, dh))
    rows_per_sub = n // _NUM_SUBCORES
    zrow = jnp.zeros((rows_per_sub, dh), jnp.float32)
    zcnt = jnp.zeros((rows_per_sub, 8), jnp.float32)
    ones = jnp.ones((_CHUNK, 8), jnp.float32)

    part, cnt = _sc_aggregate(n, d, e)(xr, src2, dst, zrow, zcnt, ones)

    bl = 1000
    grid = (n // bl,)
    xw = pl.pallas_call(
        _tc_self_body,
        grid=grid,
        in_specs=[
            pl.BlockSpec((bl, d), lambda i: (i, 0)),
            pl.BlockSpec((d, d), lambda i: (0, 0)),
            pl.BlockSpec((1, d), lambda i: (0, 0)),
        ],
        out_specs=pl.BlockSpec((bl, d), lambda i: (i, 0)),
        out_shape=jax.ShapeDtypeStruct((n, d), jnp.float32),
    )(x, W_self, b.reshape(1, d).astype(jnp.float32))

    out = pl.pallas_call(
        _tc_finish_body,
        grid=grid,
        in_specs=[
            pl.BlockSpec((bl, d), lambda i: (i, 0)),
            pl.BlockSpec((bl, d), lambda i: (i, 0)),
            pl.BlockSpec((_NUM_CORES, bl, 8), lambda i: (0, i, 0)),
            pl.BlockSpec((d, d), lambda i: (0, 0)),
        ],
        out_specs=pl.BlockSpec((bl, d), lambda i: (i, 0)),
        out_shape=jax.ShapeDtypeStruct((n, d), jnp.float32),
    )(xw, part, cnt, W_neigh)
    return out
